# baseline (speedup 1.0000x reference)
import functools

import jax
import jax.numpy as jnp
from jax import lax
from jax.experimental import pallas as pl
from jax.experimental.pallas import tpu as pltpu
from jax.experimental.pallas import tpu_sc as plsc

NUM_CODES = 128
LANES = 16
NBUF = 2
UNROLL = 16


@functools.lru_cache(maxsize=None)
def _build_sc_kernel(n_total: int, chunk: int):
    info = plsc.get_sparse_core_info()
    nc, ns = info.num_cores, info.num_subcores
    nw = nc * ns
    per_w = n_total // nw
    assert n_total % nw == 0 and per_w % chunk == 0
    n_chunks = per_w // chunk
    assert n_chunks % NBUF == 0
    mesh = plsc.VectorSubcoreMesh(core_axis_name="c", subcore_axis_name="s")

    def body(x_hbm, codes_hbm, out_hbm, codes_v, in_v, out_v, *sems):
        in_sems, out_sems = sems[:NBUF], sems[NBUF:]
        wid = lax.axis_index("s") * nc + lax.axis_index("c")
        base_w = wid * per_w
        pltpu.sync_copy(codes_hbm, codes_v)

        def start_in(j, b):
            pltpu.async_copy(
                x_hbm.at[pl.ds(base_w + j * chunk, chunk)],
                in_v.at[pl.ds(b * chunk, chunk)], in_sems[b])

        def wait_in(b):
            pltpu.make_async_copy(
                x_hbm.at[pl.ds(base_w, chunk)],
                in_v.at[pl.ds(b * chunk, chunk)], in_sems[b]).wait()

        def start_out(j, b):
            pltpu.async_copy(
                out_v.at[pl.ds(b * chunk, chunk)],
                out_hbm.at[pl.ds(base_w + j * chunk, chunk)], out_sems[b])

        def wait_out(b):
            pltpu.make_async_copy(
                out_v.at[pl.ds(b * chunk, chunk)],
                out_hbm.at[pl.ds(base_w, chunk)], out_sems[b]).wait()

        for b in range(NBUF):
            start_in(b, b)

        @pl.loop(0, n_chunks, step=NBUF)
        def _(g):
            for b in range(NBUF):
                j = g + b
                wait_in(b)

                @pl.when(j >= NBUF)
                def _():
                    wait_out(b)

                boff = b * chunk

                @plsc.parallel_loop(0, chunk // LANES, unroll=UNROLL)
                def _(i):
                    xv = in_v[pl.ds(boff + i * LANES, LANES)]
                    xc = jnp.minimum(jnp.maximum(xv, 0.0),
                                     float(NUM_CODES - 1))
                    il = xc.astype(jnp.int32)
                    ir = jnp.minimum(il + 1, NUM_CODES - 1)
                    gl = plsc.load_gather(codes_v, [il])
                    gr = plsc.load_gather(codes_v, [ir])
                    ilf = il.astype(jnp.float32)
                    out_v[pl.ds(boff + i * LANES, LANES)] = (
                        gl * (1.0 - xv + ilf) + gr * (xv - ilf))

                start_out(j, b)

                @pl.when(j + NBUF < n_chunks)
                def _():
                    start_in(j + NBUF, b)

        for b in range(NBUF):
            wait_out(b)

    return pl.kernel(
        body,
        out_type=jax.ShapeDtypeStruct((n_total,), jnp.float32),
        mesh=mesh,
        scratch_types=[
            pltpu.VMEM((NUM_CODES,), jnp.float32),
            pltpu.VMEM((NBUF * chunk,), jnp.float32),
            pltpu.VMEM((NBUF * chunk,), jnp.float32),
        ] + [pltpu.SemaphoreType.DMA] * (2 * NBUF),
        compiler_params=pltpu.CompilerParams(needs_layout_passes=False),
    )


def kernel(x, codes):
    shape = x.shape
    n_total = x.size
    x_flat = x.reshape(n_total)
    codes_flat = codes.reshape(NUM_CODES)
    chunk = 16384
    while n_total % (32 * chunk * NBUF) != 0:
        chunk //= 2
    out = _build_sc_kernel(n_total, chunk)(x_flat, codes_flat)
    return out.reshape(shape)


# speedup vs baseline: 3.2617x; 3.2617x over previous
import functools

import jax
import jax.numpy as jnp
from jax import lax
from jax.experimental import pallas as pl
from jax.experimental.pallas import tpu as pltpu
from jax.experimental.pallas import tpu_sc as plsc

NUM_CODES = 128
LANES = 16
NBUF = 2
UNROLL = 8
ROWS_PER_CHUNK = 32


@functools.lru_cache(maxsize=None)
def _build_sc_kernel(B, H, V, L):
    info = plsc.get_sparse_core_info()
    nc, ns = info.num_cores, info.num_subcores
    nw = nc * ns
    assert B * H == nw and L % LANES == 0
    cr = ROWS_PER_CHUNK
    assert V % (cr * NBUF) == 0 and cr % 8 == 0
    n_chunks = V // cr
    vregs_per_row = L // LANES
    vregs_per_chunk = cr * vregs_per_row
    mesh = plsc.VectorSubcoreMesh(core_axis_name="c", subcore_axis_name="s")

    def body(x_hbm, codes_hbm, out_hbm, codes_v, diff_v, in_v, out_v, *sems):
        in_sems, out_sems = sems[:NBUF], sems[NBUF:]
        wid = lax.axis_index("s") * nc + lax.axis_index("c")
        pltpu.sync_copy(codes_hbm, codes_v)
        for k in range(NUM_CODES // LANES):
            idx = lax.iota(jnp.int32, LANES) + (k * LANES)
            ip1 = jnp.minimum(idx + 1, NUM_CODES - 1)
            c0 = codes_v[pl.ds(k * LANES, LANES)]
            c1 = plsc.load_gather(codes_v, [ip1])
            diff_v[pl.ds(k * LANES, LANES)] = c1 - c0

        def run_plane():
            b = wid // H
            h = wid % H

            def start_in(j, buf):
                pltpu.async_copy(
                    x_hbm.at[b, h, pl.ds(j * cr, cr), :],
                    in_v.at[pl.ds(buf * cr, cr), :], in_sems[buf])

            def wait_in(buf):
                pltpu.make_async_copy(
                    x_hbm.at[b, h, pl.ds(0, cr), :],
                    in_v.at[pl.ds(buf * cr, cr), :], in_sems[buf]).wait()

            def start_out(j, buf):
                pltpu.async_copy(
                    out_v.at[pl.ds(buf * cr, cr), :],
                    out_hbm.at[b, h, pl.ds(j * cr, cr), :], out_sems[buf])

            def wait_out(buf):
                pltpu.make_async_copy(
                    out_v.at[pl.ds(buf * cr, cr), :],
                    out_hbm.at[b, h, pl.ds(0, cr), :], out_sems[buf]).wait()

            for buf in range(NBUF):
                start_in(buf, buf)

            @pl.loop(0, n_chunks, step=NBUF)
            def _(g):
                for buf in range(NBUF):
                    j = g + buf
                    wait_in(buf)

                    @pl.when(j >= NBUF)
                    def _():
                        wait_out(buf)

                    row0 = buf * cr

                    @plsc.parallel_loop(0, vregs_per_chunk, unroll=UNROLL)
                    def _(i):
                        r = row0 + i // vregs_per_row
                        cc = (i % vregs_per_row) * LANES
                        xv = in_v[r, pl.ds(cc, LANES)]
                        xc = jnp.minimum(jnp.maximum(xv, 0.0),
                                         float(NUM_CODES - 1))
                        il = xc.astype(jnp.int32)
                        gl = plsc.load_gather(codes_v, [il])
                        d = plsc.load_gather(diff_v, [il])
                        ilf = il.astype(jnp.float32)
                        out_v[r, pl.ds(cc, LANES)] = gl + d * (xv - ilf)

                    start_out(j, buf)

                    @pl.when(j + NBUF < n_chunks)
                    def _():
                        start_in(j + NBUF, buf)

        run_plane()
        for buf in range(NBUF):
            pltpu.make_async_copy(
                out_v.at[pl.ds(buf * cr, cr), :],
                out_hbm.at[0, 0, pl.ds(0, cr), :], out_sems[buf]).wait()

    return pl.kernel(
        body,
        out_type=jax.ShapeDtypeStruct((B, H, V, L), jnp.float32),
        mesh=mesh,
        scratch_types=[
            pltpu.VMEM((NUM_CODES,), jnp.float32),
            pltpu.VMEM((NUM_CODES,), jnp.float32),
            pltpu.VMEM((NBUF * cr, L), jnp.float32),
            pltpu.VMEM((NBUF * cr, L), jnp.float32),
        ] + [pltpu.SemaphoreType.DMA] * (2 * NBUF),
        compiler_params=pltpu.CompilerParams(
            needs_layout_passes=False, use_tc_tiling_on_sc=True),
    )


def kernel(x, codes):
    B, H, V, L = x.shape
    codes_flat = codes.reshape(NUM_CODES)
    return _build_sc_kernel(B, H, V, L)(x, codes_flat)
